# baseline (speedup 1.0000x reference)
import functools

import jax
import jax.numpy as jnp
from jax import lax
from jax.experimental import pallas as pl
from jax.experimental.pallas import tpu as pltpu
from jax.experimental.pallas import tpu_sc as plsc

VOCAB = 1000000
EMB = 64
BATCH = 16384
CTX = 5
LANES = 16

_info = plsc.get_sparse_core_info()
_NC, _NS = _info.num_cores, _info.num_subcores
NW = _NC * _NS
BPW = BATCH // NW
CB = 128
NCHUNK = BPW // CB


def _sc_body(t_hbm, c_hbm, ttab, ctab, out_hbm,
             tidx_v, cidx_v, trows_v, crows_v, out_v, sem):
    wid = lax.axis_index("s") * _NC + lax.axis_index("c")
    lane = lax.iota(jnp.int32, LANES)

    for chunk in range(NCHUNK):
        base = wid * BPW + chunk * CB
        pltpu.sync_copy(t_hbm.at[pl.ds(base, CB)], tidx_v)
        for j in range(CTX):
            pltpu.sync_copy(c_hbm.at[pl.ds(base * CTX + j * CB, CB)],
                            cidx_v.at[j])
        copies = [pltpu.async_copy(ttab.at[tidx_v], trows_v, sem)]
        for j in range(CTX):
            copies.append(
                pltpu.async_copy(ctab.at[cidx_v.at[j]],
                                 crows_v.at[pl.ds(j * CB, CB)], sem))
        for cp in copies:
            cp.wait()

        for g in range(CB // LANES):
            wrow = g * LANES + lane
            crows = [wrow * CTX + c for c in range(CTX)]

            def body(e, accs, wrow=wrow, crows=crows):
                col = jnp.full((LANES,), e, jnp.int32)
                w = plsc.load_gather(trows_v, [wrow, col])
                return tuple(
                    accs[c] + w * plsc.load_gather(crows_v, [crows[c], col])
                    for c in range(CTX))

            accs = lax.fori_loop(
                0, EMB, body,
                tuple(jnp.zeros((LANES,), jnp.float32) for _ in range(CTX)))
            for c in range(CTX):
                plsc.store_scatter(out_v, [crows[c]], accs[c])

        pltpu.sync_copy(out_v, out_hbm.at[pl.ds(base * CTX, CB * CTX)])


def kernel(target, context, target_table, context_table):
    t = target.reshape(BATCH).astype(jnp.int32)
    c = context.reshape(BATCH * CTX).astype(jnp.int32)

    run = functools.partial(
        pl.kernel,
        out_type=jax.ShapeDtypeStruct((BATCH * CTX,), jnp.float32),
        mesh=plsc.VectorSubcoreMesh(core_axis_name="c", subcore_axis_name="s"),
        compiler_params=pltpu.CompilerParams(
            needs_layout_passes=False, use_tc_tiling_on_sc=False),
        scratch_types=[
            pltpu.VMEM((CB,), jnp.int32),
            pltpu.VMEM((CTX, CB), jnp.int32),
            pltpu.VMEM((CB, EMB), jnp.float32),
            pltpu.VMEM((CB * CTX, EMB), jnp.float32),
            pltpu.VMEM((CB * CTX,), jnp.float32),
            pltpu.SemaphoreType.DMA,
        ],
    )(_sc_body)
    dots = run(t, c, target_table, context_table)
    return dots.reshape(BATCH, CTX)
